# baseline (speedup 1.0000x reference)
import functools

import jax
import jax.numpy as jnp
from jax import lax
from jax.experimental import pallas as pl
from jax.experimental.pallas import tpu as pltpu
from jax.experimental.pallas import tpu_sc as plsc

_NC = 2
_NS = 16
_NW = _NC * _NS
_CHUNK = 32


def _sc_copy(x_hbm, o_hbm, buf, sem):
    wid = lax.axis_index("s") * _NC + lax.axis_index("c")
    rows_w = x_hbm.shape[0] // _NW
    base = wid * rows_w
    for g in range(rows_w // _CHUNK):
        off = base + g * _CHUNK
        pltpu.async_copy(x_hbm.at[pl.ds(off, _CHUNK)], buf, sem).wait()
        pltpu.async_copy(buf, o_hbm.at[pl.ds(off, _CHUNK)], sem).wait()


def kernel(x):
    b, s, d = x.shape
    rows = b * s
    x2 = x.reshape(rows, d)
    mesh = plsc.VectorSubcoreMesh(core_axis_name="c", subcore_axis_name="s")
    f = functools.partial(
        pl.kernel,
        mesh=mesh,
        out_type=jax.ShapeDtypeStruct((rows, d), x.dtype),
        scratch_types=[
            pltpu.VMEM((_CHUNK, d), x.dtype),
            pltpu.SemaphoreType.DMA,
        ],
    )(_sc_copy)
    out = f(x2)
    return out.reshape(b, s, d)


# speedup vs baseline: 1.0700x; 1.0700x over previous
import functools

import jax
import jax.numpy as jnp
from jax import lax
from jax.experimental import pallas as pl
from jax.experimental.pallas import tpu as pltpu
from jax.experimental.pallas import tpu_sc as plsc

_NC = 2
_NS = 16
_NW = _NC * _NS
_CHUNK = 16
_NBUF = 3


def _sc_copy(x_hbm, o_hbm, bufs, rsem, wsem):
    wid = lax.axis_index("s") * _NC + lax.axis_index("c")
    rows_w = x_hbm.shape[0] // _NW
    base = wid * rows_w
    nch = rows_w // _CHUNK

    def rd(g):
        return pltpu.make_async_copy(
            x_hbm.at[pl.ds(base + g * _CHUNK, _CHUNK)],
            bufs.at[g % _NBUF], rsem.at[g % _NBUF])

    def wr(g):
        return pltpu.make_async_copy(
            bufs.at[g % _NBUF],
            o_hbm.at[pl.ds(base + g * _CHUNK, _CHUNK)], wsem.at[g % _NBUF])

    rd(0).start()
    rd(1).start()
    for g in range(nch):
        rd(g).wait()
        wr(g).start()
        nxt = g + 2
        if nxt < nch:
            if nxt - _NBUF >= 0:
                wr(nxt - _NBUF).wait()
            rd(nxt).start()
    for g in range(nch - _NBUF, nch):
        wr(g).wait()


def kernel(x):
    b, s, d = x.shape
    rows = b * s
    x2 = x.reshape(rows, d)
    mesh = plsc.VectorSubcoreMesh(core_axis_name="c", subcore_axis_name="s")
    f = functools.partial(
        pl.kernel,
        mesh=mesh,
        out_type=jax.ShapeDtypeStruct((rows, d), x.dtype),
        scratch_types=[
            pltpu.VMEM((_NBUF, _CHUNK, d), x.dtype),
            pltpu.SemaphoreType.DMA((_NBUF,)),
            pltpu.SemaphoreType.DMA((_NBUF,)),
        ],
    )(_sc_copy)
    out = f(x2)
    return out.reshape(b, s, d)


# speedup vs baseline: 1.4618x; 1.3662x over previous
import jax
import jax.numpy as jnp
from jax.experimental import pallas as pl
from jax.experimental.pallas import tpu as pltpu

_CHUNK = 512
_NBUF = 4


def _copy_ring(x_ref, o_ref, bufs, rsem, wsem):
    nch = x_ref.shape[0] // _CHUNK

    def rd(g):
        return pltpu.make_async_copy(
            x_ref.at[pl.ds(g * _CHUNK, _CHUNK)],
            bufs.at[g % _NBUF], rsem.at[g % _NBUF])

    def wr(g):
        return pltpu.make_async_copy(
            bufs.at[g % _NBUF],
            o_ref.at[pl.ds(g * _CHUNK, _CHUNK)], wsem.at[g % _NBUF])

    rd(0).start()
    rd(1).start()
    for g in range(nch):
        rd(g).wait()
        wr(g).start()
        nxt = g + 2
        if nxt < nch:
            if nxt - _NBUF >= 0:
                wr(nxt - _NBUF).wait()
            rd(nxt).start()
    for g in range(max(nch - _NBUF, 0), nch):
        wr(g).wait()


def kernel(x):
    b, s, d = x.shape
    rows = b * s
    x2 = x.reshape(rows, d)
    out = pl.pallas_call(
        _copy_ring,
        in_specs=[pl.BlockSpec(memory_space=pl.ANY)],
        out_specs=pl.BlockSpec(memory_space=pl.ANY),
        out_shape=jax.ShapeDtypeStruct((rows, d), x.dtype),
        scratch_shapes=[
            pltpu.VMEM((_NBUF, _CHUNK, d), x.dtype),
            pltpu.SemaphoreType.DMA((_NBUF,)),
            pltpu.SemaphoreType.DMA((_NBUF,)),
        ],
    )(x2)
    return out.reshape(b, s, d)


# speedup vs baseline: 1.4694x; 1.0052x over previous
import jax
import jax.numpy as jnp
from jax.experimental import pallas as pl
from jax.experimental.pallas import tpu as pltpu

_BLOCK_ROWS = 1024


def _copy_block(x_ref, o_ref):
    o_ref[...] = x_ref[...]


def kernel(x):
    b, s, d = x.shape
    rows = b * s
    x2 = x.reshape(rows, d)
    out = pl.pallas_call(
        _copy_block,
        grid=(rows // _BLOCK_ROWS,),
        in_specs=[pl.BlockSpec((_BLOCK_ROWS, d), lambda i: (i, 0))],
        out_specs=pl.BlockSpec((_BLOCK_ROWS, d), lambda i: (i, 0)),
        out_shape=jax.ShapeDtypeStruct((rows, d), x.dtype),
        compiler_params=pltpu.CompilerParams(
            dimension_semantics=("parallel",),
        ),
    )(x2)
    return out.reshape(b, s, d)
